# Initial kernel scaffold; baseline (speedup 1.0000x reference)
import jax
import jax.numpy as jnp
from jax.experimental import pallas as pl


def kernel(x, table):
    raise NotImplementedError("write your pallas kernel here")


# speedup vs baseline: 6.5688x; 6.5688x over previous
import functools

import jax
import jax.numpy as jnp
from jax import lax
from jax.experimental import pallas as pl
from jax.experimental.pallas import tpu as pltpu
from jax.experimental.pallas import tpu_sc as plsc

VOCAB1 = 100001
EMB = 64
B = 4096
L = 50

NC, NS = 2, 16
NW = NC * NS
RPW = B // NW
CB = 2
G = RPW // CB
LANES = 16
EC = EMB // LANES

_mesh = plsc.VectorSubcoreMesh(
    core_axis_name="c", subcore_axis_name="s", num_cores=NC, num_subcores=NS
)


@functools.partial(
    pl.kernel,
    out_type=jax.ShapeDtypeStruct((NW, RPW, EMB), jnp.float32),
    mesh=_mesh,
    compiler_params=pltpu.CompilerParams(use_tc_tiling_on_sc=False),
    scratch_types=[
        pltpu.VMEM((G, CB * L), jnp.int32),
        pltpu.VMEM((CB * L, EMB), jnp.float32),
        pltpu.VMEM((RPW, EMB), jnp.float32),
        pltpu.SemaphoreType.DMA,
    ],
)
def _encode(x_hbm, table_hbm, out_hbm, idx_v, rows_v, out_v, sem):
    wid = lax.axis_index("s") * NC + lax.axis_index("c")
    pltpu.sync_copy(x_hbm.at[wid], idx_v)

    scale = jnp.float32(1.0 / L)

    @pl.loop(0, G)
    def _chunk(g):
        pltpu.async_copy(table_hbm.at[idx_v.at[g]], rows_v, sem).wait()
        for r in range(CB):
            def body(l, accs):
                return tuple(
                    accs[c] + rows_v[r * L + l, pl.ds(c * LANES, LANES)]
                    for c in range(EC)
                )
            accs = lax.fori_loop(
                0, L, body, tuple(jnp.zeros((LANES,), jnp.float32) for _ in range(EC))
            )
            for c in range(EC):
                out_v[g * CB + r, pl.ds(c * LANES, LANES)] = accs[c] * scale

    pltpu.sync_copy(out_v, out_hbm.at[wid])


def kernel(x, table):
    xr = x.reshape(NW, G, CB * L)
    out = _encode(xr, table)
    return out.reshape(B, EMB)


# speedup vs baseline: 8.4192x; 1.2817x over previous
import functools

import jax
import jax.numpy as jnp
from jax import lax
from jax.experimental import pallas as pl
from jax.experimental.pallas import tpu as pltpu
from jax.experimental.pallas import tpu_sc as plsc

VOCAB1 = 100001
EMB = 64
B = 4096
L = 50

NC, NS = 2, 16
NW = NC * NS
RPW = B // NW
CB = 2
G = RPW // CB
LANES = 16
EC = EMB // LANES

_mesh = plsc.VectorSubcoreMesh(
    core_axis_name="c", subcore_axis_name="s", num_cores=NC, num_subcores=NS
)


@functools.partial(
    pl.kernel,
    out_type=jax.ShapeDtypeStruct((NW, RPW, EMB), jnp.float32),
    mesh=_mesh,
    compiler_params=pltpu.CompilerParams(use_tc_tiling_on_sc=False),
    scratch_types=[
        pltpu.VMEM((G, CB * L), jnp.int32),
        pltpu.VMEM((CB * L, EMB), jnp.float32),
        pltpu.VMEM((CB * L, EMB), jnp.float32),
        pltpu.VMEM((RPW, EMB), jnp.float32),
        pltpu.SemaphoreType.DMA,
        pltpu.SemaphoreType.DMA,
    ],
)
def _encode(x_hbm, table_hbm, out_hbm, idx_v, rows0, rows1, out_v, sem0, sem1):
    wid = lax.axis_index("s") * NC + lax.axis_index("c")
    pltpu.sync_copy(x_hbm.at[wid], idx_v)

    scale = jnp.float32(1.0 / L)
    bufs = (rows0, rows1)
    sems = (sem0, sem1)

    def compute(g, buf):
        for r in range(CB):
            accs = [buf[r * L, pl.ds(c * LANES, LANES)] for c in range(EC)]
            for l in range(1, L):
                for c in range(EC):
                    accs[c] = accs[c] + buf[r * L + l, pl.ds(c * LANES, LANES)]
            for c in range(EC):
                out_v[g * CB + r, pl.ds(c * LANES, LANES)] = accs[c] * scale

    pltpu.async_copy(table_hbm.at[idx_v.at[0]], bufs[0], sems[0])

    @pl.loop(0, G, step=2)
    def _pair(g):
        for b in range(2):
            gg = g + b

            @pl.when(gg + 1 < G)
            def _():
                pltpu.async_copy(
                    table_hbm.at[idx_v.at[gg + 1]], bufs[1 - b], sems[1 - b]
                )

            pltpu.make_async_copy(
                table_hbm.at[idx_v.at[gg]], bufs[b], sems[b]
            ).wait()
            compute(gg, bufs[b])

    pltpu.sync_copy(out_v, out_hbm.at[wid])


def kernel(x, table):
    xr = x.reshape(NW, G, CB * L)
    out = _encode(xr, table)
    return out.reshape(B, EMB)
